# Initial kernel scaffold; baseline (speedup 1.0000x reference)
import jax
import jax.numpy as jnp
from jax.experimental import pallas as pl


def kernel(x, gate_params, expert_params):
    raise NotImplementedError("write your pallas kernel here")


# speedup vs baseline: 2.2179x; 2.2179x over previous
import functools

import jax
import jax.numpy as jnp
from jax.experimental import pallas as pl

_PATCH_SIZES = [4, 8, 12, 16, 24, 32, 48, 64]
_NUM_EXPERTS = 8


def _gelu(v):
    return 0.5 * v * (1.0 + jax.lax.erf(v * 0.7071067811865476))


def _moe_kernel(x_ref, gW1_ref, gb1_ref, gW2_ref, gb2_ref,
                iW1_ref, ib1_ref, iW2_ref, ib2_ref,
                rW1_ref, rb1_ref, rW2_ref, rb2_ref,
                wW1_ref, wb1_ref, wW2_ref, wb2_ref,
                out_ref, *, T):
    X = x_ref[0]
    D = X.shape[1]
    ctx = jnp.mean(X, axis=0, keepdims=True)

    minall = jnp.min(X, axis=0, keepdims=True)
    maxall = jnp.max(X, axis=0, keepdims=True)
    feats = [ctx]
    pmeans = []
    x3s = []
    offs = []
    off = 0
    for ps in _PATCH_SIZES:
        pad = (-T) % ps
        Tp = T + pad
        N = Tp // ps
        if pad:
            Xp = jnp.concatenate([X, jnp.zeros((pad, D), X.dtype)], axis=0)
        else:
            Xp = X
        X3 = Xp.reshape(N, ps, D)
        pm = jnp.mean(X3, axis=1)
        d = X3 - pm[:, None, :]
        var = jnp.sum(d * d, axis=1) * (1.0 / (ps - 1))
        stdmean = jnp.mean(jnp.sqrt(var), axis=0, keepdims=True)
        if pad:
            fmin = jnp.minimum(minall, 0.0)
            fmax = jnp.maximum(maxall, 0.0)
        else:
            fmin, fmax = minall, maxall
        feats.extend([fmin, stdmean, fmax])
        pmeans.append(pm)
        x3s.append(X3)
        offs.append(off)
        off += ps

    gate_in = jnp.concatenate(feats, axis=1)
    hg = _gelu(jnp.dot(gate_in, gW1_ref[...], preferred_element_type=jnp.float32)
               + gb1_ref[...])
    logits = (jnp.dot(hg, gW2_ref[...], preferred_element_type=jnp.float32)
              + gb2_ref[...])

    iota = jax.lax.broadcasted_iota(jnp.int32, logits.shape, 1)
    m1 = jnp.max(logits)
    i1 = jnp.min(jnp.where(logits == m1, iota, _NUM_EXPERTS))
    rest = jnp.where(iota == i1, -3e38, logits)
    m2 = jnp.max(rest)
    i2 = jnp.min(jnp.where(rest == m2, iota, _NUM_EXPERTS))
    g1 = 1.0 / (1.0 + jnp.exp(m2 - m1))
    g2 = 1.0 - g1
    coefs = (jnp.where(iota == i1, g1, 0.0)
             + jnp.where(iota == i2, g2, 0.0))

    out_ref[0] = X

    for e, ps in enumerate(_PATCH_SIZES):
        pad = (-T) % ps
        Tp = T + pad
        N = Tp // ps

        @pl.when(coefs[0, e] != 0.0)
        def _(e=e, ps=ps, Tp=Tp, N=N, off=offs[e], pm=pmeans[e], X3=x3s[e],
              ce=coefs[0, e]):
            h = ib1_ref[e:e + 1].astype(jnp.float32)
            for p in range(ps):
                h = h + jnp.dot(X3[:, p, :], iW1_ref[off + p],
                                preferred_element_type=jnp.float32)
            hr = (jnp.dot(pm, rW1_ref[e], preferred_element_type=jnp.float32)
                  + rb1_ref[e:e + 1])
            inter = (jnp.dot(hr, rW2_ref[e], preferred_element_type=jnp.float32)
                     + rb2_ref[e:e + 1])
            hw = _gelu(jnp.dot(ctx, wW1_ref[e],
                               preferred_element_type=jnp.float32)
                       + wb1_ref[e:e + 1])
            wl = (jnp.sum(hw * wW2_ref[e:e + 1], axis=-1, keepdims=True)
                  + wb2_ref[e, 0])
            w = jax.nn.sigmoid(wl)[0, 0]
            pieces = []
            for p in range(ps):
                ip = (jnp.dot(h, iW2_ref[off + p],
                              preferred_element_type=jnp.float32)
                      + ib2_ref[off + p:off + p + 1])
                pieces.append((w * ip + (1.0 - w) * inter)[:, None, :])
            fused = jnp.concatenate(pieces, axis=1).reshape(Tp, D)[:T]
            out_ref[0] = out_ref[0] + ce * fused


@jax.jit
def kernel(x, gate_params, expert_params):
    B, T, D = x.shape
    gW1, gb1, gW2, gb2 = gate_params

    iW1s = jnp.concatenate(
        [p[0].reshape(len_ps, D, D) for p, len_ps in
         zip(expert_params, _PATCH_SIZES)], axis=0)
    ib1s = jnp.stack([p[1] for p in expert_params])
    iW2s = jnp.concatenate(
        [p[2].reshape(D, len_ps, D).transpose(1, 0, 2) for p, len_ps in
         zip(expert_params, _PATCH_SIZES)], axis=0)
    ib2s = jnp.concatenate(
        [p[3].reshape(len_ps, D) for p, len_ps in
         zip(expert_params, _PATCH_SIZES)], axis=0)
    rW1s = jnp.stack([p[4] for p in expert_params])
    rb1s = jnp.stack([p[5] for p in expert_params])
    rW2s = jnp.stack([p[6] for p in expert_params])
    rb2s = jnp.stack([p[7] for p in expert_params])
    wW1s = jnp.stack([p[8] for p in expert_params])
    wb1s = jnp.stack([p[9] for p in expert_params])
    wW2s = jnp.stack([p[10][:, 0] for p in expert_params])
    wb2s = jnp.stack([p[11] for p in expert_params])

    weights = [gW1, gb1[None, :], gW2, gb2[None, :],
               iW1s, ib1s, iW2s, ib2s,
               rW1s, rb1s, rW2s, rb2s,
               wW1s, wb1s, wW2s, wb2s]

    def full(a):
        nd = a.ndim
        return pl.BlockSpec(a.shape, lambda b, _nd=nd: (0,) * _nd)

    return pl.pallas_call(
        functools.partial(_moe_kernel, T=T),
        grid=(B,),
        in_specs=[pl.BlockSpec((1, T, D), lambda b: (b, 0, 0))]
                 + [full(a) for a in weights],
        out_specs=pl.BlockSpec((1, T, D), lambda b: (b, 0, 0)),
        out_shape=jax.ShapeDtypeStruct((B, T, D), x.dtype),
    )(x, *weights)
